# baseline (speedup 1.0000x reference)
import functools

import jax
import jax.numpy as jnp
from jax import lax
from jax.experimental import pallas as pl
from jax.experimental.pallas import tpu as pltpu
from jax.experimental.pallas import tpu_sc as plsc

D = 32
B = 4096
NL = 50
CH = 128
NC, NS = 2, 16
NW = NC * NS
DEPTH = 4
L16 = 16
G16 = CH // L16

VOCAB1 = 1000001
VPAD = 1000064
NCB = VPAD // CH
PROWS = VPAD * D // CH
CB_PER_W = -(-NCB // NW)


def _rsqrt_nr(t):
    ti = lax.bitcast_convert_type(t, jnp.int32)
    yi = jnp.int32(0x5F3759DF) - (ti >> 1)
    y = lax.bitcast_convert_type(yi, jnp.float32)
    for _ in range(2):
        y = y * (1.5 - 0.5 * t * y * y)
    return y


_DNUMS = lax.GatherDimensionNumbers(
    offset_dims=(), collapsed_slice_dims=(0,), start_index_map=(0,)
)


def _shuffle(x, perm):
    return lax.gather(x, perm[:, None], _DNUMS, (1,),
                      mode=lax.GatherScatterMode.PROMISE_IN_BOUNDS)


def _transpose16(vs):
    lane = lax.iota(jnp.int32, L16)
    for s in (1, 2, 4, 8):
        perm = lane ^ s
        keep = (lane & s) == 0
        out = list(vs)
        for i in range(L16):
            if i & s:
                continue
            lo, hi = vs[i], vs[i | s]
            out[i] = jnp.where(keep, lo, _shuffle(hi, perm))
            out[i | s] = jnp.where(keep, _shuffle(lo, perm), hi)
        vs = out
    return vs


@functools.partial(
    pl.kernel,
    mesh=plsc.VectorSubcoreMesh(core_axis_name="c", subcore_axis_name="s"),
    out_type=jax.ShapeDtypeStruct((PROWS, CH), jnp.float32),
    compiler_params=pltpu.CompilerParams(use_tc_tiling_on_sc=True),
    scratch_types=[
        pltpu.VMEM((2, D, CH), jnp.float32),
        pltpu.VMEM((2, D, CH), jnp.float32),
        pltpu.SemaphoreType.DMA,
        pltpu.SemaphoreType.DMA,
    ],
)
def _sc_pack(tT_hbm, out_hbm, in_v, fbuf, gsem, osem):
    wid = lax.axis_index("s") * NC + lax.axis_index("c")

    def cb(t):
        return wid + NW * t

    def start_load(t, b):
        pltpu.async_copy(
            tT_hbm.at[pl.ds(0, D), pl.ds(cb(t) * CH, CH)], in_v.at[b], gsem
        )

    @pl.when(cb(0) < NCB)
    def _():
        start_load(0, 0)

    def step(u, carry):
        for b in range(2):
            t = u * 2 + b

            @pl.when(cb(t) < NCB)
            def _():
                pltpu.make_async_copy(
                    tT_hbm.at[pl.ds(0, D), pl.ds(0, CH)], in_v.at[b], gsem
                ).wait()

                @pl.when(t >= 2)
                def _():
                    pltpu.make_async_copy(
                        out_hbm.at[pl.ds(0, D)], fbuf.at[b], osem
                    ).wait()

                for h in range(2):
                    for g in range(G16):
                        vs = [
                            in_v[b, h * L16 + dd, pl.ds(g * L16, L16)]
                            for dd in range(L16)
                        ]
                        ts = _transpose16(vs)
                        for k in range(L16):
                            i = g * L16 + k
                            fbuf[b, i >> 2,
                                 pl.ds(32 * (i & 3) + 16 * h, L16)] = ts[k]
                pltpu.async_copy(
                    fbuf.at[b], out_hbm.at[pl.ds(cb(t) * D, D)], osem
                )

            @pl.when(cb(t + 1) < NCB)
            def _():
                start_load(t + 1, 1 - b)

        return carry

    lax.fori_loop(0, (CB_PER_W + 1) // 2, step, 0)
    for b in range(2):
        pltpu.make_async_copy(
            out_hbm.at[pl.ds(0, D)], fbuf.at[b], osem
        ).wait()


@functools.partial(
    pl.kernel,
    mesh=plsc.VectorSubcoreMesh(core_axis_name="c", subcore_axis_name="s"),
    out_type=jax.ShapeDtypeStruct((NL * D * B // CH, CH), jnp.float32),
    compiler_params=pltpu.CompilerParams(use_tc_tiling_on_sc=False),
    scratch_types=[
        pltpu.VMEM((NL, CH), jnp.int32),
        pltpu.VMEM((DEPTH, CH, D), jnp.float32),
        pltpu.VMEM((DEPTH, D, CH), jnp.float32),
        pltpu.SemaphoreType.DMA,
        pltpu.SemaphoreType.DMA,
    ],
)
def _sc_embed(idxT_hbm, table_hbm, out_hbm, idx_v, rows_v, fbuf, gsem, osem):
    wid = lax.axis_index("s") * NC + lax.axis_index("c")
    b0 = wid * CH
    pltpu.sync_copy(idxT_hbm.at[pl.ds(0, NL), pl.ds(b0, CH)], idx_v)

    def start_gather(l, b):
        pltpu.async_copy(table_hbm.at[idx_v.at[l]], rows_v.at[b], gsem)

    for b in range(DEPTH):
        start_gather(b, b)

    def process_group(g, b):
        half = [
            [rows_v[b, g * L16 + k, pl.ds(h * L16, L16)] for k in range(L16)]
            for h in range(2)
        ]
        t0 = _transpose16(half[0])
        t1 = _transpose16(half[1])
        acc = t0[0] * t0[0]
        for t in t0[1:] + t1:
            acc = acc + t * t
        y = jnp.minimum(1.0, _rsqrt_nr(acc))
        for d in range(L16):
            fbuf[b, d, pl.ds(g * L16, L16)] = t0[d] * y
            fbuf[b, d + L16, pl.ds(g * L16, L16)] = t1[d] * y
        return b

    def step(l, carry):
        b = l & (DEPTH - 1)
        pltpu.make_async_copy(
            table_hbm.at[pl.ds(0, CH)], rows_v.at[b], gsem
        ).wait()
        @pl.when(l >= DEPTH)
        def _():
            pltpu.make_async_copy(
                out_hbm.at[pl.ds(0, D)], fbuf.at[b], osem
            ).wait()

        lax.fori_loop(0, G16, process_group, b)
        for t in range(4):
            pltpu.async_copy(
                fbuf.at[b].at[pl.ds(t * 8, 8)],
                out_hbm.at[pl.ds(l * 1024 + t * 256 + wid * 8, 8)],
                osem,
            )

        @pl.when(l + DEPTH < NL)
        def _():
            start_gather(l + DEPTH, b)

        return carry

    lax.fori_loop(0, NL, step, 0)
    for b in range(DEPTH):
        pltpu.make_async_copy(
            out_hbm.at[pl.ds(0, D)], fbuf.at[b], osem
        ).wait()


def kernel(indices, positions, table):
    tT = table.T
    packed = _sc_pack(tT)
    tbl = packed.reshape(VPAD, D)
    idxT = indices.T
    outT = _sc_embed(idxT, tbl)
    features = (outT.reshape(NL, 4, NW, 8, CH)
                .transpose(2, 4, 0, 1, 3).reshape(B, NL, D))
    mask = indices == 0
    positions_out = positions.reshape(B, NL, 1, 2)
    return positions_out, features, mask


# speedup vs baseline: 1.7517x; 1.7517x over previous
import functools

import jax
import jax.numpy as jnp
from jax import lax
from jax.experimental import pallas as pl
from jax.experimental.pallas import tpu as pltpu
from jax.experimental.pallas import tpu_sc as plsc

D = 32
B = 4096
NL = 50
CH = 128
NC, NS = 2, 16
NW = NC * NS
DEPTH = 4
L16 = 16
G16 = CH // L16

VOCAB1 = 1000001
VPAD = 1000064
NCB = VPAD // CH
PROWS = VPAD * D // CH
CB_PER_W = -(-NCB // NW)


def _rsqrt_nr(t):
    ti = lax.bitcast_convert_type(t, jnp.int32)
    yi = jnp.int32(0x5F3759DF) - (ti >> 1)
    y = lax.bitcast_convert_type(yi, jnp.float32)
    for _ in range(2):
        y = y * (1.5 - 0.5 * t * y * y)
    return y


_DNUMS = lax.GatherDimensionNumbers(
    offset_dims=(), collapsed_slice_dims=(0,), start_index_map=(0,)
)


def _shuffle(x, perm):
    return lax.gather(x, perm[:, None], _DNUMS, (1,),
                      mode=lax.GatherScatterMode.PROMISE_IN_BOUNDS)


def _transpose16(vs):
    lane = lax.iota(jnp.int32, L16)
    for s in (1, 2, 4, 8):
        perm = lane ^ s
        keep = (lane & s) == 0
        out = list(vs)
        for i in range(L16):
            if i & s:
                continue
            lo, hi = vs[i], vs[i | s]
            out[i] = jnp.where(keep, lo, _shuffle(hi, perm))
            out[i | s] = jnp.where(keep, _shuffle(lo, perm), hi)
        vs = out
    return vs


@functools.partial(
    pl.kernel,
    mesh=plsc.VectorSubcoreMesh(core_axis_name="c", subcore_axis_name="s"),
    out_type=jax.ShapeDtypeStruct((PROWS, CH), jnp.float32),
    compiler_params=pltpu.CompilerParams(use_tc_tiling_on_sc=True),
    scratch_types=[
        pltpu.VMEM((2, D, CH), jnp.float32),
        pltpu.VMEM((2, D, CH), jnp.float32),
        pltpu.SemaphoreType.DMA,
        pltpu.SemaphoreType.DMA,
    ],
)
def _sc_pack(tT_hbm, out_hbm, in_v, fbuf, gsem, osem):
    wid = lax.axis_index("s") * NC + lax.axis_index("c")

    def cb(t):
        return wid + NW * t

    def start_load(t, b):
        pltpu.async_copy(
            tT_hbm.at[pl.ds(0, D), pl.ds(cb(t) * CH, CH)], in_v.at[b], gsem
        )

    @pl.when(cb(0) < NCB)
    def _():
        start_load(0, 0)

    def step(t, carry):
        b = t & 1

        @pl.when(cb(t) < NCB)
        def _():
            pltpu.make_async_copy(
                tT_hbm.at[pl.ds(0, D), pl.ds(0, CH)], in_v.at[b], gsem
            ).wait()

            @pl.when(cb(t + 1) < NCB)
            def _():
                start_load(t + 1, 1 - b)

            @pl.when(t >= 2)
            def _():
                pltpu.make_async_copy(
                    out_hbm.at[pl.ds(0, D)], fbuf.at[b], osem
                ).wait()

            for h in range(2):
                for g in range(G16):
                    vs = [
                        in_v[b, h * L16 + dd, pl.ds(g * L16, L16)]
                        for dd in range(L16)
                    ]
                    ts = _transpose16(vs)
                    for k in range(L16):
                        i = g * L16 + k
                        fbuf[b, i >> 2, pl.ds(32 * (i & 3) + 16 * h, L16)] = ts[k]
            pltpu.async_copy(
                fbuf.at[b], out_hbm.at[pl.ds(cb(t) * D, D)], osem
            )

        return carry

    lax.fori_loop(0, CB_PER_W, step, 0)
    for b in range(2):
        pltpu.make_async_copy(
            out_hbm.at[pl.ds(0, D)], fbuf.at[b], osem
        ).wait()


@functools.partial(
    pl.kernel,
    mesh=plsc.VectorSubcoreMesh(core_axis_name="c", subcore_axis_name="s"),
    out_type=jax.ShapeDtypeStruct((NL * D * B // CH, CH), jnp.float32),
    compiler_params=pltpu.CompilerParams(use_tc_tiling_on_sc=False),
    scratch_types=[
        pltpu.VMEM((NL, CH), jnp.int32),
        pltpu.VMEM((DEPTH, CH, D), jnp.float32),
        pltpu.VMEM((DEPTH, D, CH), jnp.float32),
        pltpu.SemaphoreType.DMA,
        pltpu.SemaphoreType.DMA,
    ],
)
def _sc_embed(idxT_hbm, table_hbm, out_hbm, idx_v, rows_v, fbuf, gsem, osem):
    wid = lax.axis_index("s") * NC + lax.axis_index("c")
    b0 = wid * CH
    pltpu.sync_copy(idxT_hbm.at[pl.ds(0, NL), pl.ds(b0, CH)], idx_v)

    def start_gather(l, b):
        pltpu.async_copy(table_hbm.at[idx_v.at[l]], rows_v.at[b], gsem)

    for b in range(DEPTH):
        start_gather(b, b)

    def process_group(g, b):
        half = [
            [rows_v[b, g * L16 + k, pl.ds(h * L16, L16)] for k in range(L16)]
            for h in range(2)
        ]
        t0 = _transpose16(half[0])
        t1 = _transpose16(half[1])
        acc = t0[0] * t0[0]
        for t in t0[1:] + t1:
            acc = acc + t * t
        y = jnp.minimum(1.0, _rsqrt_nr(acc))
        for d in range(L16):
            fbuf[b, d, pl.ds(g * L16, L16)] = t0[d] * y
            fbuf[b, d + L16, pl.ds(g * L16, L16)] = t1[d] * y
        return b

    def step(l, carry):
        b = l & (DEPTH - 1)
        pltpu.make_async_copy(
            table_hbm.at[pl.ds(0, CH)], rows_v.at[b], gsem
        ).wait()
        @pl.when(l >= DEPTH)
        def _():
            pltpu.make_async_copy(
                out_hbm.at[pl.ds(0, D)], fbuf.at[b], osem
            ).wait()

        lax.fori_loop(0, G16, process_group, b)
        for t in range(4):
            pltpu.async_copy(
                fbuf.at[b].at[pl.ds(t * 8, 8)],
                out_hbm.at[pl.ds(l * 1024 + t * 256 + wid * 8, 8)],
                osem,
            )

        @pl.when(l + DEPTH < NL)
        def _():
            start_gather(l + DEPTH, b)

        return carry

    lax.fori_loop(0, NL, step, 0)
    for b in range(DEPTH):
        pltpu.make_async_copy(
            out_hbm.at[pl.ds(0, D)], fbuf.at[b], osem
        ).wait()


def kernel(indices, positions, table):
    tT = table.T
    packed = _sc_pack(tT)
    tbl = packed.reshape(VPAD, D)
    idxT = indices.T
    outT = _sc_embed(idxT, tbl)
    features = (outT.reshape(NL, 4, NW, 8, CH)
                .transpose(2, 4, 0, 1, 3).reshape(B, NL, D))
    mask = indices == 0
    positions_out = positions.reshape(B, NL, 1, 2)
    return positions_out, features, mask
